# baseline (speedup 1.0000x reference)
import functools

import jax
import jax.numpy as jnp
from jax import lax
from jax.experimental import pallas as pl
from jax.experimental.pallas import tpu as pltpu
from jax.experimental.pallas import tpu_sc as plsc

_D = 128
_B = 512
_NW = 32
_BW = _B // _NW


def _sc_partials(class_emb, rel_emb, i10, i11, i20, i21, i22, i30, i3r, i32c):
    mesh = plsc.VectorSubcoreMesh(core_axis_name="c", subcore_axis_name="s")
    f32 = jnp.float32

    @functools.partial(
        pl.kernel,
        mesh=mesh,
        compiler_params=pltpu.CompilerParams(
            use_tc_tiling_on_sc=False, needs_layout_passes=False),
        out_type=(
            jax.ShapeDtypeStruct((_B,), f32),
            jax.ShapeDtypeStruct((_B,), f32),
            jax.ShapeDtypeStruct((_B,), f32),
            jax.ShapeDtypeStruct((_B,), f32),
        ),
        scratch_types=(
            [pltpu.VMEM((_BW,), jnp.int32) for _ in range(8)]
            + [pltpu.VMEM((_BW, 2 * _D), f32) for _ in range(7)]
            + [pltpu.VMEM((_BW, _D), f32)]
            + [pltpu.VMEM((_BW,), f32) for _ in range(4)]
            + [pltpu.SemaphoreType.DMA for _ in range(8)]
        ),
    )
    def k(class_hbm, rel_hbm, c10, c11, c20, c21, c22, c30, c3r, c32,
          o1, oa, ob, o3,
          x10, x11, x20, x21, x22, x30, x3r, x32,
          v1c, v1d, v2c, v2d, v2e, v3c, v3d, v3r,
          s1v, sav, sbv, s3v,
          m1c, m1d, m2c, m2d, m2e, m3c, m3d, m3r):
        wid = lax.axis_index("s") * 2 + lax.axis_index("c")
        base = wid * _BW
        sl = pl.ds(base, _BW)

        pltpu.sync_copy(c10.at[sl], x10)
        pltpu.sync_copy(c11.at[sl], x11)
        pltpu.sync_copy(c20.at[sl], x20)
        pltpu.sync_copy(c21.at[sl], x21)
        pltpu.sync_copy(c22.at[sl], x22)
        pltpu.sync_copy(c30.at[sl], x30)
        pltpu.sync_copy(c3r.at[sl], x3r)
        pltpu.sync_copy(c32.at[sl], x32)

        cp1c = pltpu.async_copy(class_hbm.at[x10], v1c, m1c)
        cp1d = pltpu.async_copy(class_hbm.at[x11], v1d, m1d)
        cp2c = pltpu.async_copy(class_hbm.at[x20], v2c, m2c)
        cp2d = pltpu.async_copy(class_hbm.at[x21], v2d, m2d)
        cp2e = pltpu.async_copy(class_hbm.at[x22], v2e, m2e)
        cp3c = pltpu.async_copy(class_hbm.at[x30], v3c, m3c)
        cp3d = pltpu.async_copy(class_hbm.at[x32], v3d, m3d)
        cp3r = pltpu.async_copy(rel_hbm.at[x3r], v3r, m3r)

        lanes = lax.broadcasted_iota(jnp.int32, (_BW,), 0)

        def col(ref, dvec):
            return plsc.load_gather(ref, [lanes, dvec])

        cp1c.wait()
        cp1d.wait()

        def body1(d, acc):
            dc = jnp.full((_BW,), d, jnp.int32)
            dc2 = dc + _D
            c1 = col(v1c, dc)
            cr = col(v1c, dc2)
            d1 = col(v1d, dc)
            dr = col(v1d, dc2)
            t = jnp.maximum(jnp.abs(c1 - d1) + jnp.abs(cr) - jnp.abs(dr), 0.0)
            return acc + t * t

        s1v[...] = lax.fori_loop(0, _D, body1, jnp.zeros((_BW,), f32))
        pltpu.sync_copy(s1v, o1.at[sl])

        cp2c.wait()
        cp2d.wait()
        cp2e.wait()

        def body2(d, accs):
            aa, ab = accs
            dc = jnp.full((_BW,), d, jnp.int32)
            dc2 = dc + _D
            c1 = col(v2c, dc)
            c2 = jnp.abs(col(v2c, dc2))
            d1 = col(v2d, dc)
            d2 = jnp.abs(col(v2d, dc2))
            e1 = col(v2e, dc)
            e2 = jnp.abs(col(v2e, dc2))
            start = jnp.maximum(c1 - c2, d1 - d2)
            end = jnp.minimum(c1 + c2, d1 + d2)
            diff = start - end
            cen = (start + end) * 0.5
            t1 = jnp.maximum(jnp.abs(cen - e1) + jnp.abs(diff) * 0.5 - e2, 0.0)
            t2 = jnp.maximum(diff, 0.0)
            return (aa + t1 * t1, ab + t2 * t2)

        ra, rb = lax.fori_loop(
            0, _D, body2, (jnp.zeros((_BW,), f32), jnp.zeros((_BW,), f32)))
        sav[...] = ra
        sbv[...] = rb
        pltpu.sync_copy(sav, oa.at[sl])
        pltpu.sync_copy(sbv, ob.at[sl])

        cp3c.wait()
        cp3d.wait()
        cp3r.wait()

        def body3(d, acc):
            dc = jnp.full((_BW,), d, jnp.int32)
            dc2 = dc + _D
            c1 = col(v3c, dc)
            cr = col(v3c, dc2)
            d1 = col(v3d, dc)
            dr = col(v3d, dc2)
            r = col(v3r, dc)
            t = jnp.maximum(jnp.abs(c1 + r - d1) + jnp.abs(cr) - jnp.abs(dr), 0.0)
            return acc + t * t

        s3v[...] = lax.fori_loop(0, _D, body3, jnp.zeros((_BW,), f32))
        pltpu.sync_copy(s3v, o3.at[sl])

    return k(class_emb, rel_emb, i10, i11, i20, i21, i22, i30, i3r, i32c)


def _reduce_body(s1_ref, sa_ref, sb_ref, s3_ref, o_ref):
    s1 = s1_ref[...]
    sa = sa_ref[...]
    sb = sb_ref[...]
    s3 = s3_ref[...]
    a = jnp.sqrt(sa)
    b = jnp.sqrt(sb)
    inv = 1.0 / _B
    loss = (jnp.sum(s1) + jnp.sum(sa) + jnp.sum(sb) + jnp.sum(s3)) * inv \
        + 2.0 * (jnp.sum(a) * inv) * (jnp.sum(b) * inv)
    o_ref[...] = jnp.full((1, 1), loss, jnp.float32)


def kernel(class_emb, rel_emb, nf1, nf2, nf3):
    i32 = jnp.int32
    s1, sa, sb, s3 = _sc_partials(
        class_emb, rel_emb,
        nf1[:, 0].astype(i32), nf1[:, 1].astype(i32),
        nf2[:, 0].astype(i32), nf2[:, 1].astype(i32), nf2[:, 2].astype(i32),
        nf3[:, 0].astype(i32), nf3[:, 1].astype(i32), nf3[:, 2].astype(i32),
    )
    out = pl.pallas_call(
        _reduce_body,
        out_shape=jax.ShapeDtypeStruct((1, 1), jnp.float32),
    )(s1.reshape(4, 128), sa.reshape(4, 128),
      sb.reshape(4, 128), s3.reshape(4, 128))
    return out[0, 0]


# speedup vs baseline: 1.0317x; 1.0317x over previous
import functools

import jax
import jax.numpy as jnp
from jax import lax
from jax.experimental import pallas as pl
from jax.experimental.pallas import tpu as pltpu
from jax.experimental.pallas import tpu_sc as plsc

_D = 128
_B = 512
_NW = 32
_BW = _B // _NW


def _sc_partials(class_emb, rel_emb, i10, i11, i20, i21, i22, i30, i3r, i32c):
    mesh = plsc.VectorSubcoreMesh(core_axis_name="c", subcore_axis_name="s")
    f32 = jnp.float32

    @functools.partial(
        pl.kernel,
        mesh=mesh,
        compiler_params=pltpu.CompilerParams(
            use_tc_tiling_on_sc=False, needs_layout_passes=False),
        out_type=(
            jax.ShapeDtypeStruct((_B,), f32),
            jax.ShapeDtypeStruct((_B,), f32),
            jax.ShapeDtypeStruct((_B,), f32),
            jax.ShapeDtypeStruct((_B,), f32),
        ),
        scratch_types=(
            [pltpu.VMEM((_BW,), jnp.int32) for _ in range(8)]
            + [pltpu.VMEM((_BW, 2 * _D), f32) for _ in range(7)]
            + [pltpu.VMEM((_BW, _D), f32)]
            + [pltpu.VMEM((_BW,), f32) for _ in range(4)]
            + [pltpu.SemaphoreType.DMA for _ in range(8)]
        ),
    )
    def k(class_hbm, rel_hbm, c10, c11, c20, c21, c22, c30, c3r, c32,
          o1, oa, ob, o3,
          x10, x11, x20, x21, x22, x30, x3r, x32,
          v1c, v1d, v2c, v2d, v2e, v3c, v3d, v3r,
          s1v, sav, sbv, s3v,
          m1c, m1d, m2c, m2d, m2e, m3c, m3d, m3r):
        wid = lax.axis_index("s") * 2 + lax.axis_index("c")
        base = wid * _BW
        sl = pl.ds(base, _BW)

        pltpu.sync_copy(c10.at[sl], x10)
        pltpu.sync_copy(c11.at[sl], x11)
        pltpu.sync_copy(c20.at[sl], x20)
        pltpu.sync_copy(c21.at[sl], x21)
        pltpu.sync_copy(c22.at[sl], x22)
        pltpu.sync_copy(c30.at[sl], x30)
        pltpu.sync_copy(c3r.at[sl], x3r)
        pltpu.sync_copy(c32.at[sl], x32)

        cp1c = pltpu.async_copy(class_hbm.at[x10], v1c, m1c)
        cp1d = pltpu.async_copy(class_hbm.at[x11], v1d, m1d)
        cp2c = pltpu.async_copy(class_hbm.at[x20], v2c, m2c)
        cp2d = pltpu.async_copy(class_hbm.at[x21], v2d, m2d)
        cp2e = pltpu.async_copy(class_hbm.at[x22], v2e, m2e)
        cp3c = pltpu.async_copy(class_hbm.at[x30], v3c, m3c)
        cp3d = pltpu.async_copy(class_hbm.at[x32], v3d, m3d)
        cp3r = pltpu.async_copy(rel_hbm.at[x3r], v3r, m3r)

        lanes = lax.broadcasted_iota(jnp.int32, (_BW,), 0)

        def col(ref, dvec):
            return plsc.load_gather(ref, [lanes, dvec])

        cp1c.wait()
        cp1d.wait()

        def body1(d, acc):
            dc = jnp.full((_BW,), d, jnp.int32)
            dc2 = dc + _D
            c1 = col(v1c, dc)
            cr = col(v1c, dc2)
            d1 = col(v1d, dc)
            dr = col(v1d, dc2)
            t = jnp.maximum(jnp.abs(c1 - d1) + jnp.abs(cr) - jnp.abs(dr), 0.0)
            return acc + t * t

        s1v[...] = plsc.parallel_loop(
            0, _D, unroll=8, carry=jnp.zeros((_BW,), f32))(body1)
        pltpu.sync_copy(s1v, o1.at[sl])

        cp2c.wait()
        cp2d.wait()
        cp2e.wait()

        def body2(d, accs):
            aa, ab = accs
            dc = jnp.full((_BW,), d, jnp.int32)
            dc2 = dc + _D
            c1 = col(v2c, dc)
            c2 = jnp.abs(col(v2c, dc2))
            d1 = col(v2d, dc)
            d2 = jnp.abs(col(v2d, dc2))
            e1 = col(v2e, dc)
            e2 = jnp.abs(col(v2e, dc2))
            start = jnp.maximum(c1 - c2, d1 - d2)
            end = jnp.minimum(c1 + c2, d1 + d2)
            diff = start - end
            cen = (start + end) * 0.5
            t1 = jnp.maximum(jnp.abs(cen - e1) + jnp.abs(diff) * 0.5 - e2, 0.0)
            t2 = jnp.maximum(diff, 0.0)
            return (aa + t1 * t1, ab + t2 * t2)

        ra, rb = plsc.parallel_loop(
            0, _D, unroll=8,
            carry=(jnp.zeros((_BW,), f32), jnp.zeros((_BW,), f32)))(body2)
        sav[...] = ra
        sbv[...] = rb
        pltpu.sync_copy(sav, oa.at[sl])
        pltpu.sync_copy(sbv, ob.at[sl])

        cp3c.wait()
        cp3d.wait()
        cp3r.wait()

        def body3(d, acc):
            dc = jnp.full((_BW,), d, jnp.int32)
            dc2 = dc + _D
            c1 = col(v3c, dc)
            cr = col(v3c, dc2)
            d1 = col(v3d, dc)
            dr = col(v3d, dc2)
            r = col(v3r, dc)
            t = jnp.maximum(jnp.abs(c1 + r - d1) + jnp.abs(cr) - jnp.abs(dr), 0.0)
            return acc + t * t

        s3v[...] = plsc.parallel_loop(
            0, _D, unroll=8, carry=jnp.zeros((_BW,), f32))(body3)
        pltpu.sync_copy(s3v, o3.at[sl])

    return k(class_emb, rel_emb, i10, i11, i20, i21, i22, i30, i3r, i32c)


def _reduce_body(s1_ref, sa_ref, sb_ref, s3_ref, o_ref):
    s1 = s1_ref[...]
    sa = sa_ref[...]
    sb = sb_ref[...]
    s3 = s3_ref[...]
    a = jnp.sqrt(sa)
    b = jnp.sqrt(sb)
    inv = 1.0 / _B
    loss = (jnp.sum(s1) + jnp.sum(sa) + jnp.sum(sb) + jnp.sum(s3)) * inv \
        + 2.0 * (jnp.sum(a) * inv) * (jnp.sum(b) * inv)
    o_ref[...] = jnp.full((1, 1), loss, jnp.float32)


def kernel(class_emb, rel_emb, nf1, nf2, nf3):
    i32 = jnp.int32
    s1, sa, sb, s3 = _sc_partials(
        class_emb, rel_emb,
        nf1[:, 0].astype(i32), nf1[:, 1].astype(i32),
        nf2[:, 0].astype(i32), nf2[:, 1].astype(i32), nf2[:, 2].astype(i32),
        nf3[:, 0].astype(i32), nf3[:, 1].astype(i32), nf3[:, 2].astype(i32),
    )
    out = pl.pallas_call(
        _reduce_body,
        out_shape=jax.ShapeDtypeStruct((1, 1), jnp.float32),
    )(s1.reshape(4, 128), sa.reshape(4, 128),
      sb.reshape(4, 128), s3.reshape(4, 128))
    return out[0, 0]
